# baseline (speedup 1.0000x reference)
import math

import jax
import jax.numpy as jnp
from jax.experimental import pallas as pl


BLOCK_L = 512


def _pe_add_kernel(x_ref, w_ref, o_ref):
    w = w_ref[...]
    d_model = w.shape[-1]
    max_norm = math.sqrt(float(d_model))
    norm_sq = jnp.sum(w * w, axis=-1, keepdims=True)
    norm = jnp.sqrt(norm_sq)
    scale = jnp.minimum(1.0, max_norm / jnp.maximum(norm, 1e-12))
    o_ref[...] = x_ref[...] + w * scale


def kernel(x, W):
    batch, seq_len, d_model = x.shape
    block_l = min(BLOCK_L, seq_len)
    num_l = seq_len // block_l
    xf = x.reshape(batch * seq_len, d_model)
    out = pl.pallas_call(
        _pe_add_kernel,
        grid=(num_l, batch),
        in_specs=[
            pl.BlockSpec((block_l, d_model), lambda l, b: (b * num_l + l, 0)),
            pl.BlockSpec((block_l, d_model), lambda l, b: (l, 0)),
        ],
        out_specs=pl.BlockSpec((block_l, d_model), lambda l, b: (b * num_l + l, 0)),
        out_shape=jax.ShapeDtypeStruct(xf.shape, xf.dtype),
    )(xf, W)
    return out.reshape(x.shape)


# speedup vs baseline: 1.3144x; 1.3144x over previous
import math

import jax
import jax.numpy as jnp
from jax.experimental import pallas as pl


BLOCK_L = 1024


def _pe_add_kernel(x_ref, w_ref, o_ref):
    w = w_ref[...]
    d_model = w.shape[-1]
    max_norm = math.sqrt(float(d_model))
    norm_sq = jnp.sum(w * w, axis=-1, keepdims=True)
    norm = jnp.sqrt(norm_sq)
    scale = jnp.minimum(1.0, max_norm / jnp.maximum(norm, 1e-12))
    o_ref[...] = x_ref[...] + w * scale


def _pe_add_kernel3(x_ref, w_ref, o_ref):
    w = w_ref[...]
    d_model = w.shape[-1]
    max_norm = math.sqrt(float(d_model))
    norm_sq = jnp.sum(w * w, axis=-1, keepdims=True)
    norm = jnp.sqrt(norm_sq)
    scale = jnp.minimum(1.0, max_norm / jnp.maximum(norm, 1e-12))
    o_ref[...] = x_ref[...] + (w * scale)[None, :, :]


def kernel(x, W):
    batch, seq_len, d_model = x.shape
    block_l = min(BLOCK_L, seq_len)
    grid = (seq_len // block_l,)
    return pl.pallas_call(
        _pe_add_kernel3,
        grid=grid,
        in_specs=[
            pl.BlockSpec((batch, block_l, d_model), lambda i: (0, i, 0)),
            pl.BlockSpec((block_l, d_model), lambda i: (i, 0)),
        ],
        out_specs=pl.BlockSpec((batch, block_l, d_model), lambda i: (0, i, 0)),
        out_shape=jax.ShapeDtypeStruct(x.shape, x.dtype),
    )(x, W)
